# baseline (device time: 342252 ns/iter reference)
import jax
import jax.numpy as jnp
from jax import lax
from jax.experimental import pallas as pl
from jax.experimental.pallas import tpu as pltpu

N_DEV = 4
M_PER = 2048
K = 8192
N_PER = 1024
BN = 512
KH = K // 2
XCH = 128
WCH = 1024

N_XPHASE = 16
N_SUB = 16
P_TOT = N_XPHASE + N_SUB


def kernel(x, w_mat):
    my_pos = lax.axis_index("i")
    targets = (my_pos + jnp.array([1, 2, 3, 0], dtype=jnp.int32)) % N_DEV

    def body(tgt_ref, x_ref, w_ref, out_ref, rbuf,
             xb, xstage, wb, acc, sbuf, send_sems, recv_sems, copy_sems):
        p = pl.program_id(0)
        c = pl.program_id(1)
        my = lax.axis_index("i")
        barrier = pltpu.get_barrier_semaphore()

        @pl.when((p == 0) & (c == 0))
        def _():
            for off in range(1, N_DEV):
                pl.semaphore_signal(
                    barrier, inc=1,
                    device_id=((my + off) % N_DEV,),
                    device_id_type=pl.DeviceIdType.MESH,
                )
            pl.semaphore_wait(barrier, N_DEV - 1)

        @pl.when(p < N_XPHASE)
        def _():
            i = p * 4 + c
            xstage[...] = x_ref[...].astype(jnp.bfloat16)
            half = i // 32
            col = (i % 32) * XCH
            cp = pltpu.make_async_copy(
                xstage,
                xb.at[half, :, pl.ds(col, XCH)],
                copy_sems.at[0],
            )
            cp.start()
            cp.wait()

        @pl.when((p >= N_XPHASE - 1) & (p < P_TOT - 1))
        def _():
            q_next = p - (N_XPHASE - 1)
            wb[q_next % 2, pl.ds(c * WCH, WCH), :] = (
                w_ref[...].astype(jnp.bfloat16))

        @pl.when((p >= N_XPHASE) & (c == 3))
        def _():
            q = p - N_XPHASE
            t_idx = q // 4
            subcol = (q // 2) % 2
            half = q % 2

            res = jnp.dot(
                xb[half],
                wb[q % 2],
                preferred_element_type=jnp.float32,
            )

            @pl.when(half == 0)
            def _():
                acc[...] = res

            @pl.when(half == 1)
            def _():
                acc[...] += res

                @pl.when(t_idx < 3)
                def _():
                    slot = t_idx % 2

                    @pl.when((t_idx == 2) & (subcol == 0))
                    def _():
                        prev = pltpu.make_async_remote_copy(
                            src_ref=sbuf.at[0],
                            dst_ref=rbuf.at[0],
                            send_sem=send_sems.at[0],
                            recv_sem=recv_sems.at[my],
                            device_id=(tgt_ref[0],),
                            device_id_type=pl.DeviceIdType.MESH,
                        )
                        prev.wait_send()

                    @pl.when(subcol == 0)
                    def _():
                        sbuf[slot, :, 0:BN] = acc[...].astype(jnp.bfloat16)

                    @pl.when(subcol == 1)
                    def _():
                        sbuf[slot, :, BN:N_PER] = acc[...].astype(jnp.bfloat16)
                        rdma = pltpu.make_async_remote_copy(
                            src_ref=sbuf.at[slot],
                            dst_ref=rbuf.at[N_DEV - 2 - t_idx],
                            send_sem=send_sems.at[t_idx],
                            recv_sem=recv_sems.at[my],
                            device_id=(tgt_ref[t_idx],),
                            device_id_type=pl.DeviceIdType.MESH,
                        )
                        rdma.start()

                @pl.when(t_idx == 3)
                def _():
                    cp = pltpu.make_async_copy(
                        acc,
                        out_ref.at[pl.ds(my * M_PER, M_PER),
                                   pl.ds(subcol * BN, BN)],
                        copy_sems.at[1],
                    )
                    cp.start()
                    cp.wait()

        @pl.when((p == P_TOT - 1) & (c == 3))
        def _():
            for i in (1, 2):
                snd = pltpu.make_async_remote_copy(
                    src_ref=sbuf.at[i % 2],
                    dst_ref=rbuf.at[0],
                    send_sem=send_sems.at[i],
                    recv_sem=recv_sems.at[my],
                    device_id=((my + 1 + i) % N_DEV,),
                    device_id_type=pl.DeviceIdType.MESH,
                )
                snd.wait_send()

            for j in range(N_DEV - 1):
                src = (my - 1 - j) % N_DEV
                slot = N_DEV - 2 - j
                recv = pltpu.make_async_remote_copy(
                    src_ref=sbuf.at[0],
                    dst_ref=rbuf.at[slot],
                    send_sem=send_sems.at[0],
                    recv_sem=recv_sems.at[src],
                    device_id=(src,),
                    device_id_type=pl.DeviceIdType.MESH,
                )
                recv.wait_recv()
                bcp = pltpu.make_async_copy(
                    rbuf.at[slot], sbuf.at[0], copy_sems.at[2])
                bcp.start()
                bcp.wait()
                for cc in range(2):
                    acc[...] = sbuf[0, :, cc * BN:(cc + 1) * BN].astype(
                        jnp.float32)
                    cp = pltpu.make_async_copy(
                        acc,
                        out_ref.at[pl.ds(src * M_PER, M_PER),
                                   pl.ds(cc * BN, BN)],
                        copy_sems.at[3],
                    )
                    cp.start()
                    cp.wait()

    def x_map(p, c, tgt):
        return (0, jnp.where(p < N_XPHASE, p * 4 + c, 63))

    def w_map(p, c, tgt):
        q_next = jnp.clip(p - (N_XPHASE - 1), 0, N_SUB - 1)
        row = (q_next % 2) * 4 + c
        col = tgt[q_next // 4] * 2 + (q_next // 2) % 2
        return (row, col)

    grid_spec = pltpu.PrefetchScalarGridSpec(
        num_scalar_prefetch=1,
        grid=(P_TOT, 4),
        in_specs=[
            pl.BlockSpec((M_PER, XCH), x_map),
            pl.BlockSpec((WCH, BN), w_map),
        ],
        out_specs=[
            pl.BlockSpec(memory_space=pl.ANY),
            pl.BlockSpec(memory_space=pl.ANY),
        ],
        scratch_shapes=[
            pltpu.VMEM((2, M_PER, KH), jnp.bfloat16),
            pltpu.VMEM((M_PER, XCH), jnp.bfloat16),
            pltpu.VMEM((2, KH, BN), jnp.bfloat16),
            pltpu.VMEM((M_PER, BN), jnp.float32),
            pltpu.VMEM((2, M_PER, N_PER), jnp.bfloat16),
            pltpu.SemaphoreType.DMA((N_DEV - 1,)),
            pltpu.SemaphoreType.DMA((N_DEV,)),
            pltpu.SemaphoreType.DMA((4,)),
        ],
    )

    y, _ = pl.pallas_call(
        body,
        grid_spec=grid_spec,
        out_shape=[
            jax.ShapeDtypeStruct((N_DEV * M_PER, N_PER), jnp.float32),
            jax.ShapeDtypeStruct((N_DEV - 1, M_PER, N_PER), jnp.bfloat16),
        ],
        compiler_params=pltpu.CompilerParams(
            collective_id=0,
            dimension_semantics=("arbitrary", "arbitrary"),
            vmem_limit_bytes=63 * 1024 * 1024,
        ),
    )(targets, x, w_mat)
    return y


# device time: 337814 ns/iter; 1.0131x vs baseline; 1.0131x over previous
import jax
import jax.numpy as jnp
from jax import lax
from jax.experimental import pallas as pl
from jax.experimental.pallas import tpu as pltpu

N_DEV = 4
M_PER = 2048
K = 8192
N_PER = 1024
BN = 512
KH = K // 2
XCH = 128
WCH = 1024

N_XPHASE = 16
N_SUB = 16
P_TOT = N_XPHASE + N_SUB


def kernel(x, w_mat):
    my_pos = lax.axis_index("i")
    targets = (my_pos + jnp.array([1, 2, 3, 0], dtype=jnp.int32)) % N_DEV

    def body(tgt_ref, x_ref, w_ref, out_ref, rbuf,
             xb0, xb1, xstage, wba, wbb, acc, sbuf,
             send_sems, recv_sems, copy_sems):
        p = pl.program_id(0)
        c = pl.program_id(1)
        my = lax.axis_index("i")
        barrier = pltpu.get_barrier_semaphore()

        @pl.when((p == 0) & (c == 0))
        def _():
            for off in range(1, N_DEV):
                pl.semaphore_signal(
                    barrier, inc=1,
                    device_id=((my + off) % N_DEV,),
                    device_id_type=pl.DeviceIdType.MESH,
                )
            pl.semaphore_wait(barrier, N_DEV - 1)

        @pl.when(p < N_XPHASE)
        def _():
            i = p * 4 + c
            slot = i % 2

            @pl.when(i >= 2)
            def _():
                pltpu.make_async_copy(
                    xstage.at[slot], xstage.at[slot], copy_sems.at[slot]
                ).wait()

            xstage[slot] = x_ref[...].astype(jnp.bfloat16)
            col = (i % 32) * XCH

            @pl.when(i < 32)
            def _():
                pltpu.make_async_copy(
                    xstage.at[slot], xb0.at[:, pl.ds(col, XCH)],
                    copy_sems.at[slot],
                ).start()

            @pl.when(i >= 32)
            def _():
                pltpu.make_async_copy(
                    xstage.at[slot], xb1.at[:, pl.ds(col, XCH)],
                    copy_sems.at[slot],
                ).start()

            @pl.when(i == 63)
            def _():
                pltpu.make_async_copy(
                    xstage.at[0], xstage.at[0], copy_sems.at[0]).wait()
                pltpu.make_async_copy(
                    xstage.at[1], xstage.at[1], copy_sems.at[1]).wait()

        @pl.when((p >= N_XPHASE - 1) & (p < P_TOT - 1))
        def _():
            q_next = p - (N_XPHASE - 1)

            @pl.when(q_next % 2 == 0)
            def _():
                wba[pl.ds(c * WCH, WCH), :] = w_ref[...].astype(jnp.bfloat16)

            @pl.when(q_next % 2 == 1)
            def _():
                wbb[pl.ds(c * WCH, WCH), :] = w_ref[...].astype(jnp.bfloat16)

        @pl.when((p >= N_XPHASE) & (c == 3))
        def _():
            q = p - N_XPHASE
            t_idx = q // 4
            subcol = (q // 2) % 2
            half = q % 2

            @pl.when(half == 0)
            def _():
                acc[...] = jnp.dot(
                    xb0[...], wba[...], preferred_element_type=jnp.float32)

            @pl.when(half == 1)
            def _():
                acc[...] += jnp.dot(
                    xb1[...], wbb[...], preferred_element_type=jnp.float32)

                @pl.when(t_idx < 3)
                def _():
                    slot = t_idx % 2

                    @pl.when((t_idx == 2) & (subcol == 0))
                    def _():
                        prev = pltpu.make_async_remote_copy(
                            src_ref=sbuf.at[0],
                            dst_ref=rbuf.at[0],
                            send_sem=send_sems.at[0],
                            recv_sem=recv_sems.at[my],
                            device_id=(tgt_ref[0],),
                            device_id_type=pl.DeviceIdType.MESH,
                        )
                        prev.wait_send()

                    @pl.when(subcol == 0)
                    def _():
                        sbuf[slot, :, 0:BN] = acc[...].astype(jnp.bfloat16)

                    @pl.when(subcol == 1)
                    def _():
                        sbuf[slot, :, BN:N_PER] = acc[...].astype(jnp.bfloat16)
                        rdma = pltpu.make_async_remote_copy(
                            src_ref=sbuf.at[slot],
                            dst_ref=rbuf.at[N_DEV - 2 - t_idx],
                            send_sem=send_sems.at[t_idx],
                            recv_sem=recv_sems.at[my],
                            device_id=(tgt_ref[t_idx],),
                            device_id_type=pl.DeviceIdType.MESH,
                        )
                        rdma.start()

                @pl.when(t_idx == 3)
                def _():
                    cp = pltpu.make_async_copy(
                        acc,
                        out_ref.at[pl.ds(my * M_PER, M_PER),
                                   pl.ds(subcol * BN, BN)],
                        copy_sems.at[1],
                    )
                    cp.start()
                    cp.wait()

        @pl.when((p == P_TOT - 1) & (c == 3))
        def _():
            for i in (1, 2):
                snd = pltpu.make_async_remote_copy(
                    src_ref=sbuf.at[i % 2],
                    dst_ref=rbuf.at[0],
                    send_sem=send_sems.at[i],
                    recv_sem=recv_sems.at[my],
                    device_id=((my + 1 + i) % N_DEV,),
                    device_id_type=pl.DeviceIdType.MESH,
                )
                snd.wait_send()

            for j in range(N_DEV - 1):
                src = (my - 1 - j) % N_DEV
                slot = N_DEV - 2 - j
                recv = pltpu.make_async_remote_copy(
                    src_ref=sbuf.at[0],
                    dst_ref=rbuf.at[slot],
                    send_sem=send_sems.at[0],
                    recv_sem=recv_sems.at[src],
                    device_id=(src,),
                    device_id_type=pl.DeviceIdType.MESH,
                )
                recv.wait_recv()
                bcp = pltpu.make_async_copy(
                    rbuf.at[slot], sbuf.at[0], copy_sems.at[2])
                bcp.start()
                bcp.wait()
                for cc in range(2):
                    acc[...] = sbuf[0, :, cc * BN:(cc + 1) * BN].astype(
                        jnp.float32)
                    cp = pltpu.make_async_copy(
                        acc,
                        out_ref.at[pl.ds(src * M_PER, M_PER),
                                   pl.ds(cc * BN, BN)],
                        copy_sems.at[3],
                    )
                    cp.start()
                    cp.wait()

    def x_map(p, c, tgt):
        return (0, jnp.where(p < N_XPHASE, p * 4 + c, 63))

    def w_map(p, c, tgt):
        q_next = jnp.clip(p - (N_XPHASE - 1), 0, N_SUB - 1)
        row = (q_next % 2) * 4 + c
        col = tgt[q_next // 4] * 2 + (q_next // 2) % 2
        return (row, col)

    grid_spec = pltpu.PrefetchScalarGridSpec(
        num_scalar_prefetch=1,
        grid=(P_TOT, 4),
        in_specs=[
            pl.BlockSpec((M_PER, XCH), x_map),
            pl.BlockSpec((WCH, BN), w_map),
        ],
        out_specs=[
            pl.BlockSpec(memory_space=pl.ANY),
            pl.BlockSpec(memory_space=pl.ANY),
        ],
        scratch_shapes=[
            pltpu.VMEM((M_PER, KH), jnp.bfloat16),
            pltpu.VMEM((M_PER, KH), jnp.bfloat16),
            pltpu.VMEM((2, M_PER, XCH), jnp.bfloat16),
            pltpu.VMEM((KH, BN), jnp.bfloat16),
            pltpu.VMEM((KH, BN), jnp.bfloat16),
            pltpu.VMEM((M_PER, BN), jnp.float32),
            pltpu.VMEM((2, M_PER, N_PER), jnp.bfloat16),
            pltpu.SemaphoreType.DMA((N_DEV - 1,)),
            pltpu.SemaphoreType.DMA((N_DEV,)),
            pltpu.SemaphoreType.DMA((4,)),
        ],
    )

    y, _ = pl.pallas_call(
        body,
        grid_spec=grid_spec,
        out_shape=[
            jax.ShapeDtypeStruct((N_DEV * M_PER, N_PER), jnp.float32),
            jax.ShapeDtypeStruct((N_DEV - 1, M_PER, N_PER), jnp.bfloat16),
        ],
        compiler_params=pltpu.CompilerParams(
            collective_id=0,
            dimension_semantics=("arbitrary", "arbitrary"),
            vmem_limit_bytes=63 * 1024 * 1024,
        ),
    )(targets, x, w_mat)
    return y


# device time: 227136 ns/iter; 1.5068x vs baseline; 1.4873x over previous
import jax
import jax.numpy as jnp
from jax import lax
from jax.experimental import pallas as pl
from jax.experimental.pallas import tpu as pltpu

N_DEV = 4
M_PER = 2048
K = 8192
N_PER = 1024
BK = 1024
NK = K // BK


def kernel(x, w_mat):
    my_pos = lax.axis_index("i")
    targets = (my_pos + jnp.array([1, 2, 3, 0], dtype=jnp.int32)) % N_DEV

    def body(tgt_ref, x_ref, w_ref, out_ref,
             acc, sbuf, rbuf, send_sems, recv_sems, copy_sems):
        s = pl.program_id(0)
        k = pl.program_id(1)
        my = lax.axis_index("i")
        barrier = pltpu.get_barrier_semaphore()

        @pl.when((s == 0) & (k == 0))
        def _():
            for off in range(1, N_DEV):
                pl.semaphore_signal(
                    barrier, inc=1,
                    device_id=((my + off) % N_DEV,),
                    device_id_type=pl.DeviceIdType.MESH,
                )
            pl.semaphore_wait(barrier, N_DEV - 1)

        partial = jnp.dot(
            x_ref[...].astype(jnp.bfloat16),
            w_ref[...].astype(jnp.bfloat16),
            preferred_element_type=jnp.float32,
        )

        @pl.when(k == 0)
        def _():
            acc[...] = partial

        @pl.when(k > 0)
        def _():
            acc[...] += partial

        @pl.when(k == NK - 1)
        def _():
            @pl.when(s < N_DEV - 1)
            def _():
                slot = s % 2
                @pl.when(s == 2)
                def _():
                    prev = pltpu.make_async_remote_copy(
                        src_ref=sbuf.at[0],
                        dst_ref=rbuf.at[0],
                        send_sem=send_sems.at[0],
                        recv_sem=recv_sems.at[my],
                        device_id=(tgt_ref[0],),
                        device_id_type=pl.DeviceIdType.MESH,
                    )
                    prev.wait_send()

                sbuf[slot] = acc[...].astype(jnp.bfloat16)
                rdma = pltpu.make_async_remote_copy(
                    src_ref=sbuf.at[slot],
                    dst_ref=rbuf.at[N_DEV - 2 - s],
                    send_sem=send_sems.at[s],
                    recv_sem=recv_sems.at[my],
                    device_id=(tgt_ref[s],),
                    device_id_type=pl.DeviceIdType.MESH,
                )
                rdma.start()

            @pl.when(s == N_DEV - 1)
            def _():
                own_cp = pltpu.make_async_copy(
                    acc,
                    out_ref.at[pl.ds(my * M_PER, M_PER), :],
                    copy_sems.at[N_DEV - 1],
                )
                own_cp.start()
                own_cp.wait()

                for j in range(N_DEV - 1):
                    src = (my - 1 - j) % N_DEV
                    slot = N_DEV - 2 - j
                    recv = pltpu.make_async_remote_copy(
                        src_ref=sbuf.at[0],
                        dst_ref=rbuf.at[slot],
                        send_sem=send_sems.at[0],
                        recv_sem=recv_sems.at[src],
                        device_id=(src,),
                        device_id_type=pl.DeviceIdType.MESH,
                    )
                    recv.wait_recv()
                    acc[...] = rbuf[slot].astype(jnp.float32)
                    cp = pltpu.make_async_copy(
                        acc,
                        out_ref.at[pl.ds(src * M_PER, M_PER), :],
                        copy_sems.at[j],
                    )
                    cp.start()
                    cp.wait()

                for i in (1, 2):
                    snd = pltpu.make_async_remote_copy(
                        src_ref=sbuf.at[i % 2],
                        dst_ref=rbuf.at[0],
                        send_sem=send_sems.at[i],
                        recv_sem=recv_sems.at[my],
                        device_id=((my + 1 + i) % N_DEV,),
                        device_id_type=pl.DeviceIdType.MESH,
                    )
                    snd.wait_send()

    grid_spec = pltpu.PrefetchScalarGridSpec(
        num_scalar_prefetch=1,
        grid=(N_DEV, NK),
        in_specs=[
            pl.BlockSpec((M_PER, BK), lambda s, k, tgt: (0, k)),
            pl.BlockSpec((BK, N_PER), lambda s, k, tgt: (k, tgt[s])),
        ],
        out_specs=pl.BlockSpec(memory_space=pl.ANY),
        scratch_shapes=[
            pltpu.VMEM((M_PER, N_PER), jnp.float32),
            pltpu.VMEM((2, M_PER, N_PER), jnp.bfloat16),
            pltpu.VMEM((N_DEV - 1, M_PER, N_PER), jnp.bfloat16),
            pltpu.SemaphoreType.DMA((N_DEV - 1,)),
            pltpu.SemaphoreType.DMA((N_DEV,)),
            pltpu.SemaphoreType.DMA((N_DEV,)),
        ],
    )

    return pl.pallas_call(
        body,
        grid_spec=grid_spec,
        out_shape=jax.ShapeDtypeStruct((N_DEV * M_PER, N_PER), jnp.float32),
        compiler_params=pltpu.CompilerParams(
            collective_id=0,
            dimension_semantics=("arbitrary", "arbitrary"),
            vmem_limit_bytes=63 * 1024 * 1024,
        ),
    )(targets, x, w_mat)
